# baseline (speedup 1.0000x reference)
import functools

import jax
import jax.numpy as jnp
from jax import lax
from jax.experimental import pallas as pl
from jax.experimental.pallas import tpu as pltpu
from jax.experimental.pallas import tpu_sc as plsc

_CW = 128
_BN = 16384


def _make_sc_gather(num_units, emb_dim, units_per_worker, table_dtype):
    mesh = plsc.VectorSubcoreMesh(core_axis_name="c", subcore_axis_name="s")
    num_cores = plsc.get_sparse_core_info().num_cores

    @functools.partial(
        pl.kernel,
        mesh=mesh,
        out_type=jax.ShapeDtypeStruct((num_units * 8, 8, _CW), table_dtype),
        scratch_types=[
            pltpu.VMEM((units_per_worker, _CW), jnp.int32),
            pltpu.VMEM((_CW, emb_dim), table_dtype),
            pltpu.VMEM((emb_dim, _CW), table_dtype),
            pltpu.SemaphoreType.DMA,
            pltpu.SemaphoreType.DMA,
        ],
        compiler_params=pltpu.CompilerParams(
            use_tc_tiling_on_sc=False, needs_layout_passes=False
        ),
    )
    def gather_k(table_hbm, idx_hbm, out_hbm, idx_v, rows_v, trows_v, sem, wsem):
        wid = lax.axis_index("s") * num_cores + lax.axis_index("c")
        pltpu.sync_copy(
            idx_hbm.at[pl.ds(wid * units_per_worker, units_per_worker)], idx_v
        )
        base_unit = wid * units_per_worker
        lanes = lax.iota(jnp.int32, 16)

        def body(j, carry):
            g = base_unit + j
            pltpu.async_copy(table_hbm.at[idx_v.at[j]], rows_v, sem).wait()

            def dbody(d, c):
                dvec = jnp.full((16,), d, jnp.int32)
                for k in range(8):
                    v = plsc.load_gather(rows_v, [lanes + (16 * k), dvec])
                    trows_v[d, pl.ds(16 * k, 16)] = v
                return c

            lax.fori_loop(0, emb_dim, dbody, 0)

            row0 = (g // _CW) * 1024 + (g % _CW)
            copies = [
                pltpu.async_copy(
                    trows_v.at[pl.ds(dg * 8, 8)], out_hbm.at[row0 + dg * _CW], wsem
                )
                for dg in range(8)
            ]
            for c in copies:
                c.wait()
            return carry

        lax.fori_loop(0, units_per_worker, body, 0)

    return gather_k


def _fuse_body(t_ref, w_ref, out_ref):
    dn = (((0,), (1,)), ((), ()))
    proj = lax.dot_general(
        t_ref[...], w_ref[...], dn, preferred_element_type=jnp.float32
    )
    out_ref[...] = jnp.concatenate(
        [proj[: _BN // 2], proj[_BN // 2 :]], axis=1
    )


def kernel(input, table, W):
    B, L = input.shape
    V, D = table.shape
    O = W.shape[0]
    n = B * L

    nblk = (V + _BN - 1) // _BN
    vpad = nblk * _BN

    idx = input.T.reshape(-1).astype(jnp.int32)
    j = idx % _BN
    ridx = (idx - j) + 2 * (j % (_BN // 2)) + j // (_BN // 2)
    ridx = ridx.reshape(n // _CW, _CW)

    tableT = table.T
    fused2 = pl.pallas_call(
        _fuse_body,
        grid=(nblk,),
        in_specs=[
            pl.BlockSpec((D, _BN), lambda i: (0, i)),
            pl.BlockSpec((O, D), lambda i: (0, 0)),
        ],
        out_specs=pl.BlockSpec((_BN // 2, 2 * O), lambda i: (i, 0)),
        out_shape=jax.ShapeDtypeStruct((vpad // 2, 2 * O), jnp.float32),
    )(tableT, W)
    fused_lin = fused2.reshape(vpad, O)

    num_units = n // _CW
    gather_k = _make_sc_gather(num_units, O, num_units // 32, fused_lin.dtype)
    out3 = gather_k(fused_lin, ridx)

    x5 = out3.reshape(L, 8, B // _CW, 8, _CW)
    r = x5.transpose(0, 1, 3, 2, 4).reshape(L, O, B)
    return r.transpose(2, 0, 1)


# speedup vs baseline: 1.1870x; 1.1870x over previous
import functools

import jax
import jax.numpy as jnp
from jax import lax
from jax.experimental import pallas as pl
from jax.experimental.pallas import tpu as pltpu
from jax.experimental.pallas import tpu_sc as plsc

_CW = 128
_BN = 16384


def _make_sc_gather(num_units, emb_dim, units_per_worker, table_dtype):
    mesh = plsc.VectorSubcoreMesh(core_axis_name="c", subcore_axis_name="s")
    num_cores = plsc.get_sparse_core_info().num_cores

    @functools.partial(
        pl.kernel,
        mesh=mesh,
        out_type=jax.ShapeDtypeStruct((num_units * 8, 8, _CW), table_dtype),
        scratch_types=[
            pltpu.VMEM((units_per_worker, _CW), jnp.int32),
            pltpu.VMEM((2, _CW, emb_dim), table_dtype),
            pltpu.VMEM((2, emb_dim, _CW), table_dtype),
            pltpu.SemaphoreType.DMA,
            pltpu.SemaphoreType.DMA,
            pltpu.SemaphoreType.DMA,
            pltpu.SemaphoreType.DMA,
        ],
        compiler_params=pltpu.CompilerParams(
            use_tc_tiling_on_sc=False, needs_layout_passes=False
        ),
    )
    def gather_k(
        table_hbm, idx_hbm, out_hbm, idx_v, rows_v, trows_v, gsem0, gsem1, wsem0, wsem1
    ):
        wid = lax.axis_index("s") * num_cores + lax.axis_index("c")
        pltpu.sync_copy(
            idx_hbm.at[pl.ds(wid * units_per_worker, units_per_worker)], idx_v
        )
        base_unit = wid * units_per_worker
        lanes = lax.iota(jnp.int32, 16)
        npairs = units_per_worker // 2
        gsems = (gsem0, gsem1)
        wsems = (wsem0, wsem1)

        def start_gather(j, buf, sem):
            pltpu.async_copy(table_hbm.at[idx_v.at[j]], rows_v.at[buf], sem)

        def wait_gather(buf, sem):
            pltpu.make_async_copy(
                table_hbm.at[idx_v.at[0]], rows_v.at[buf], sem
            ).wait()

        def drain_writes(buf, sem):
            for _ in range(8):
                pltpu.make_async_copy(
                    trows_v.at[buf].at[pl.ds(0, 8)], out_hbm.at[0], sem
                ).wait()

        def do_unit(j, buf):
            def dbody(d, c):
                dvec = jnp.full((16,), d, jnp.int32)
                for k in range(8):
                    v = plsc.load_gather(rows_v.at[buf], [lanes + (16 * k), dvec])
                    trows_v[buf, d, pl.ds(16 * k, 16)] = v
                return c

            lax.fori_loop(0, emb_dim, dbody, 0)

            g = base_unit + j
            row0 = (g // _CW) * 1024 + (g % _CW)
            for dg in range(8):
                pltpu.async_copy(
                    trows_v.at[buf].at[pl.ds(dg * 8, 8)],
                    out_hbm.at[row0 + dg * _CW],
                    wsems[buf],
                )

        start_gather(0, 0, gsems[0])

        def body(i, carry):
            ja = 2 * i
            start_gather(ja + 1, 1, gsems[1])
            wait_gather(0, gsems[0])

            @pl.when(i > 0)
            def _():
                drain_writes(0, wsems[0])

            do_unit(ja, 0)

            @pl.when(i < npairs - 1)
            def _():
                start_gather(ja + 2, 0, gsems[0])

            wait_gather(1, gsems[1])

            @pl.when(i > 0)
            def _():
                drain_writes(1, wsems[1])

            do_unit(ja + 1, 1)
            return carry

        lax.fori_loop(0, npairs, body, 0)
        drain_writes(0, wsems[0])
        drain_writes(1, wsems[1])

    return gather_k


def _fuse_body(t_ref, w_ref, out_ref):
    dn = (((0,), (1,)), ((), ()))
    proj = lax.dot_general(
        t_ref[...], w_ref[...], dn, preferred_element_type=jnp.float32
    )
    out_ref[...] = jnp.concatenate(
        [proj[: _BN // 2], proj[_BN // 2 :]], axis=1
    )


def kernel(input, table, W):
    B, L = input.shape
    V, D = table.shape
    O = W.shape[0]
    n = B * L

    nblk = (V + _BN - 1) // _BN
    vpad = nblk * _BN

    idx = input.T.reshape(-1).astype(jnp.int32)
    j = idx % _BN
    ridx = (idx - j) + 2 * (j % (_BN // 2)) + j // (_BN // 2)
    ridx = ridx.reshape(n // _CW, _CW)

    tableT = table.T
    fused2 = pl.pallas_call(
        _fuse_body,
        grid=(nblk,),
        in_specs=[
            pl.BlockSpec((D, _BN), lambda i: (0, i)),
            pl.BlockSpec((O, D), lambda i: (0, 0)),
        ],
        out_specs=pl.BlockSpec((_BN // 2, 2 * O), lambda i: (i, 0)),
        out_shape=jax.ShapeDtypeStruct((vpad // 2, 2 * O), jnp.float32),
    )(tableT, W)
    fused_lin = fused2.reshape(vpad, O)

    num_units = n // _CW
    gather_k = _make_sc_gather(num_units, O, num_units // 32, fused_lin.dtype)
    out3 = gather_k(fused_lin, ridx)

    x5 = out3.reshape(L, 8, B // _CW, 8, _CW)
    r = x5.transpose(0, 1, 3, 2, 4).reshape(L, O, B)
    return r.transpose(2, 0, 1)


# speedup vs baseline: 1.6669x; 1.4043x over previous
import functools

import jax
import jax.numpy as jnp
from jax import lax
from jax.experimental import pallas as pl
from jax.experimental.pallas import tpu as pltpu
from jax.experimental.pallas import tpu_sc as plsc

_CW = 128
_BN = 16384


def _make_sc_gather(num_rows, emb_dim, table_rows, chunks_per_worker, table_dtype):
    rows_per_worker = chunks_per_worker * _CW
    mesh = plsc.VectorSubcoreMesh(core_axis_name="c", subcore_axis_name="s")
    num_cores = plsc.get_sparse_core_info().num_cores

    @functools.partial(
        pl.kernel,
        mesh=mesh,
        out_type=jax.ShapeDtypeStruct((num_rows, emb_dim), table_dtype),
        scratch_types=[
            pltpu.VMEM((chunks_per_worker, _CW), jnp.int32),
            pltpu.VMEM((_CW, emb_dim), table_dtype),
            pltpu.SemaphoreType.DMA,
        ],
        compiler_params=pltpu.CompilerParams(use_tc_tiling_on_sc=False),
    )
    def gather_k(table_hbm, idx_hbm, out_hbm, idx_v, rows_v, sem):
        wid = lax.axis_index("s") * num_cores + lax.axis_index("c")
        pltpu.sync_copy(
            idx_hbm.at[pl.ds(wid * chunks_per_worker, chunks_per_worker)], idx_v
        )
        base_row = wid * rows_per_worker

        def body(j, carry):
            pltpu.async_copy(table_hbm.at[idx_v.at[j]], rows_v, sem).wait()
            pltpu.sync_copy(rows_v, out_hbm.at[pl.ds(base_row + j * _CW, _CW)])
            return carry

        lax.fori_loop(0, chunks_per_worker, body, 0)

    return gather_k


def _fuse_body(t_ref, w_ref, out_ref):
    dn = (((0,), (1,)), ((), ()))
    proj = lax.dot_general(
        t_ref[...], w_ref[...], dn, preferred_element_type=jnp.float32
    )
    out_ref[...] = jnp.concatenate(
        [proj[: _BN // 2], proj[_BN // 2 :]], axis=1
    )


def kernel(input, table, W):
    B, L = input.shape
    V, D = table.shape
    O = W.shape[0]
    n = B * L

    nblk = (V + _BN - 1) // _BN
    vpad = nblk * _BN

    idx = input.reshape(-1).astype(jnp.int32)
    j = idx % _BN
    ridx = (idx - j) + 2 * (j % (_BN // 2)) + j // (_BN // 2)
    ridx = ridx.reshape(n // _CW, _CW)

    tableT = table.T
    fused2 = pl.pallas_call(
        _fuse_body,
        grid=(nblk,),
        in_specs=[
            pl.BlockSpec((D, _BN), lambda i: (0, i)),
            pl.BlockSpec((O, D), lambda i: (0, 0)),
        ],
        out_specs=pl.BlockSpec((_BN // 2, 2 * O), lambda i: (i, 0)),
        out_shape=jax.ShapeDtypeStruct((vpad // 2, 2 * O), jnp.float32),
    )(tableT, W)
    fused_lin = fused2.reshape(vpad, O)

    chunks_per_worker = n // (_CW * 32)
    gather_k = _make_sc_gather(n, O, vpad, chunks_per_worker, fused_lin.dtype)
    out = gather_k(fused_lin, ridx)

    return out.reshape(B, L, O)
